# baseline (speedup 1.0000x reference)
import jax
import jax.numpy as jnp
from jax import lax
from jax.experimental import pallas as pl
from jax.experimental.pallas import tpu as pltpu

_NBUF = 4
_R = 256


def kernel(x, pos_embedding):
    B, S, D = x.shape
    nch = (B * S) // _R
    cpb = S // _R

    def body(x_hbm, pe_hbm, o_hbm, xb, ob, peb, in_sems, out_sems, pe_sem):
        def in_copy(c, slot):
            b = c // cpb
            s0 = (c % cpb) * _R
            return pltpu.make_async_copy(
                x_hbm.at[b, pl.ds(s0, _R)], xb.at[slot], in_sems.at[slot]
            )

        def out_copy(c, slot):
            b = c // cpb
            s0 = (c % cpb) * _R
            return pltpu.make_async_copy(
                ob.at[slot], o_hbm.at[b, pl.ds(s0, _R)], out_sems.at[slot]
            )

        pe_copy = pltpu.make_async_copy(pe_hbm.at[pl.ds(0, S)], peb, pe_sem)
        pe_copy.start()
        for i in range(_NBUF - 1):
            in_copy(i, i).start()
        pe_copy.wait()

        def step(c, carry):
            slot = lax.rem(c, _NBUF)
            in_copy(c, slot).wait()

            @pl.when(c >= _NBUF)
            def _():
                out_copy(c - _NBUF, slot).wait()

            s0 = (c % cpb) * _R
            ob[slot] = xb[slot] + peb[pl.ds(s0, _R), :]
            out_copy(c, slot).start()

            nc = c + _NBUF - 1

            @pl.when(nc < nch)
            def _():
                in_copy(nc, lax.rem(nc, _NBUF)).start()

            return carry

        lax.fori_loop(0, nch, step, 0)
        for i in range(_NBUF):
            c = nch - _NBUF + i
            out_copy(c, c % _NBUF).wait()

    return pl.pallas_call(
        body,
        in_specs=[
            pl.BlockSpec(memory_space=pl.ANY),
            pl.BlockSpec(memory_space=pl.ANY),
        ],
        out_specs=pl.BlockSpec(memory_space=pl.ANY),
        out_shape=jax.ShapeDtypeStruct(x.shape, x.dtype),
        scratch_shapes=[
            pltpu.VMEM((_NBUF, _R, D), jnp.float32),
            pltpu.VMEM((_NBUF, _R, D), jnp.float32),
            pltpu.VMEM((S, D), jnp.float32),
            pltpu.SemaphoreType.DMA((_NBUF,)),
            pltpu.SemaphoreType.DMA((_NBUF,)),
            pltpu.SemaphoreType.DMA,
        ],
    )(x, pos_embedding)


# speedup vs baseline: 1.0137x; 1.0137x over previous
import jax
import jax.numpy as jnp
from jax.experimental import pallas as pl


def _add_body(x_ref, pe_ref, o_ref):
    o_ref[...] = x_ref[...] + pe_ref[...]


def kernel(x, pos_embedding):
    B, S, D = x.shape
    TS = 1024
    return pl.pallas_call(
        _add_body,
        grid=(S // TS, B),
        in_specs=[
            pl.BlockSpec((1, TS, D), lambda s, b: (b, s, 0)),
            pl.BlockSpec((TS, D), lambda s, b: (s, 0)),
        ],
        out_specs=pl.BlockSpec((1, TS, D), lambda s, b: (b, s, 0)),
        out_shape=jax.ShapeDtypeStruct(x.shape, x.dtype),
    )(x, pos_embedding)
